# baseline (speedup 1.0000x reference)
import functools

import jax
import jax.numpy as jnp
from jax import lax
from jax.experimental import pallas as pl
from jax.experimental.pallas import tpu as pltpu
from jax.experimental.pallas import tpu_sc as plsc

_MAXP = 128
_D = 64
_LQ = 2048
_LK = 2048
_TFLAT = 257 * _D

_NDT = 8
_NJT = _LK // 128
_NIQ = 4
_IB = _LQ // _NIQ
_SPAN = _LK + _IB
_W = 384
_WCH = _W // 128
_M = _IB // 8


def _rp_body(table_hbm, delta_hbm, out_hbm, idx_v, slab_v, delta_v, fb_v,
             fbi_v, row_v, gsem, sem):
    wid = lax.axis_index("s") * 2 + lax.axis_index("c")
    dt = wid % _NDT
    iq = wid // _NDT
    d0 = dt * 8
    i0 = iq * _IB
    t_lo = (_LQ - _IB) - i0

    pltpu.sync_copy(delta_hbm, delta_v)
    delta = delta_v[...]
    base = t_lo - (_LQ - 1)
    lanes = lax.iota(jnp.int32, 16)

    zl = lanes * 0
    for dm in range(8):
        fbi_v[0, pl.ds(dm * 16, 16)] = zl + (d0 + dm)
        fbi_v[1, pl.ds(dm * 16, 16)] = zl + (256 * _D + d0 + dm)
    pltpu.make_async_copy(table_hbm.at[fbi_v.at[0]], fb_v.at[0], gsem).start()
    pltpu.make_async_copy(table_hbm.at[fbi_v.at[1]], fb_v.at[1], gsem).start()
    for _ in range(2):
        pltpu.make_async_copy(
            table_hbm.at[fbi_v.at[0]], fb_v.at[0], gsem).wait()
    fl = [fb_v[0, pl.ds(dm * 16, 16)] for dm in range(8)]
    fr = [fb_v[1, pl.ds(dm * 16, 16)] for dm in range(8)]

    def stage(s, pb):
        u0 = -(base + s) - _MAXP
        w0 = 8 * jnp.clip(lax.div(u0 - 60, 8), 0, (_SPAN - _W) // 8)

        for dm in range(8):
            def fill(kk, c2, dm=dm):
                t = lanes + (kk * 16 + base + s) + delta
                val = jnp.where(t <= -_MAXP, fl[dm], fr[dm])
                slab_v[pb, dm, pl.ds(kk * 16, 16)] = val
                return c2

            lax.fori_loop(0, _SPAN // 16, fill, 0)

        for dm in range(8):
            def fill_idx(q, c2, dm=dm):
                t = lanes + (q * 16 + w0 + base + s) + delta
                t = jnp.minimum(jnp.maximum(t, -_MAXP), _MAXP) + _MAXP
                idx_v[dm * _WCH + q // 8,
                      pl.ds((q % 8) * 16, 16)] = t * _D + d0 + dm
                return c2

            lax.fori_loop(0, _W // 16, fill_idx, 0)

        for g in range(8 * _WCH):
            pltpu.make_async_copy(
                table_hbm.at[idx_v.at[g]],
                slab_v.at[pb, g // _WCH, pl.ds(w0 + (g % _WCH) * 128, 128)],
                gsem,
            ).start()

    def drain_stage():
        for g in range(8 * _WCH):
            pltpu.make_async_copy(
                table_hbm.at[idx_v.at[0]],
                slab_v.at[0, 0, pl.ds(0, 128)],
                gsem,
            ).wait()

    def phase(s, carry):
        pb = lax.rem(s, 2)
        stage(s + 1, 1 - pb)

        def assemble(m, rb):
            off = 8 * ((_M - 1) - m)

            def tiles(c, c2):
                for dm in range(8):
                    for k in range(8):
                        row_v[rb, c, dm, pl.ds(k * 16, 16)] = (
                            slab_v[pb, dm, pl.ds(off + c * 128 + k * 16, 16)])
                return c2

            lax.fori_loop(0, _NJT, tiles, 0)

        def fire(m, rb):
            ri = (7 - s) + 8 * m
            pltpu.make_async_copy(
                row_v.at[rb], out_hbm.at[i0 + ri, dt], sem).start()

        def wait_one():
            pltpu.make_async_copy(
                row_v.at[0], out_hbm.at[i0, dt], sem).wait()

        assemble(0, 0)
        fire(0, 0)
        assemble(1, 1)
        fire(1, 1)

        def pump(m, c2):
            wait_one()
            rb = lax.rem(m, 2)
            assemble(m, rb)
            fire(m, rb)
            return c2

        lax.fori_loop(2, _M, pump, 0)
        wait_one()
        wait_one()

        drain_stage()
        return carry

    stage(0, 0)
    drain_stage()
    lax.fori_loop(0, 8, phase, 0)


_rp_call = functools.partial(
    pl.kernel,
    mesh=plsc.VectorSubcoreMesh(core_axis_name="c", subcore_axis_name="s"),
    out_type=jax.ShapeDtypeStruct((_LQ, _NDT, _NJT, 8, 128), jnp.float32),
    scratch_types=[
        pltpu.VMEM((8 * _WCH, 128), jnp.int32),
        pltpu.VMEM((2, 8, _SPAN), jnp.float32),
        pltpu.VMEM((16,), jnp.int32),
        pltpu.VMEM((2, 128), jnp.float32),
        pltpu.VMEM((2, 128), jnp.int32),
        pltpu.VMEM((2, _NJT, 8, 128), jnp.float32),
        pltpu.SemaphoreType.DMA,
        pltpu.SemaphoreType.DMA,
    ],
    compiler_params=pltpu.CompilerParams(use_tc_tiling_on_sc=False),
)(_rp_body)


def kernel(length_q, length_k, embeddings_table):
    tbl = embeddings_table.astype(jnp.float32).reshape(_TFLAT)
    delta = jnp.zeros((16,), jnp.int32) + (
        jnp.asarray(length_k, jnp.int32) - jnp.asarray(length_q, jnp.int32))
    out5 = _rp_call(tbl, delta)
    return out5.transpose(0, 2, 4, 1, 3).reshape(_LQ, _LK, _D)


# speedup vs baseline: 2.1013x; 2.1013x over previous
import functools

import jax
import jax.numpy as jnp
from jax import lax
from jax.experimental import pallas as pl
from jax.experimental.pallas import tpu as pltpu
from jax.experimental.pallas import tpu_sc as plsc

_MAXP = 128
_D = 64
_LQ = 2048
_LK = 2048
_TFLAT = 257 * _D

_NDT = 8
_NJT = _LK // 128
_NIQ = 4
_IB = _LQ // _NIQ
_SPAN = _LK + _IB
_W = 384
_WCH = _W // 128
_M = _IB // 8


def _rp_body(table_hbm, delta_hbm, out_hbm, idx_v, slab_v, delta_v, fb_v,
             fbi_v, row_v, gsem, sem):
    wid = lax.axis_index("s") * 2 + lax.axis_index("c")
    dt = wid % _NDT
    iq = wid // _NDT
    d0 = dt * 8
    i0 = iq * _IB
    t_lo = (_LQ - _IB) - i0

    pltpu.sync_copy(delta_hbm, delta_v)
    delta = delta_v[...]
    base = t_lo - (_LQ - 1)
    lanes = lax.iota(jnp.int32, 16)

    zl = lanes * 0
    for dm in range(8):
        fbi_v[0, pl.ds(dm * 16, 16)] = zl + (d0 + dm)
        fbi_v[1, pl.ds(dm * 16, 16)] = zl + (256 * _D + d0 + dm)
    pltpu.make_async_copy(table_hbm.at[fbi_v.at[0]], fb_v.at[0], gsem).start()
    pltpu.make_async_copy(table_hbm.at[fbi_v.at[1]], fb_v.at[1], gsem).start()
    for _ in range(2):
        pltpu.make_async_copy(
            table_hbm.at[fbi_v.at[0]], fb_v.at[0], gsem).wait()
    fl = [fb_v[0, pl.ds(dm * 16, 16)] for dm in range(8)]
    fr = [fb_v[1, pl.ds(dm * 16, 16)] for dm in range(8)]

    def stage(s, pb):
        u0 = -(base + s) - _MAXP
        w0 = 8 * jnp.clip(lax.div(u0 - 60, 8), 0, (_SPAN - _W) // 8)

        for dm in range(8):
            def fill(kk, c2, dm=dm):
                t = lanes + (kk * 16 + base + s) + delta
                val = jnp.where(t <= -_MAXP, fl[dm], fr[dm])
                slab_v[pb, dm, pl.ds(kk * 16, 16)] = val
                return c2

            lax.fori_loop(0, _SPAN // 16, fill, 0)

        for dm in range(8):
            def fill_idx(q, c2, dm=dm):
                t = lanes + (q * 16 + w0 + base + s) + delta
                t = jnp.minimum(jnp.maximum(t, -_MAXP), _MAXP) + _MAXP
                idx_v[dm * _WCH + q // 8,
                      pl.ds((q % 8) * 16, 16)] = t * _D + d0 + dm
                return c2

            lax.fori_loop(0, _W // 16, fill_idx, 0)

        for g in range(8 * _WCH):
            pltpu.make_async_copy(
                table_hbm.at[idx_v.at[g]],
                slab_v.at[pb, g // _WCH, pl.ds(w0 + (g % _WCH) * 128, 128)],
                gsem,
            ).start()

    def drain_stage():
        for g in range(8 * _WCH):
            pltpu.make_async_copy(
                table_hbm.at[idx_v.at[0]],
                slab_v.at[0, 0, pl.ds(0, 128)],
                gsem,
            ).wait()

    def phase(s, carry):
        pb = lax.rem(s, 2)
        w0_s = 8 * jnp.clip(lax.div(-(base + s) - _MAXP - 60, 8),
                            0, (_SPAN - _W) // 8)
        stage(s + 1, 1 - pb)

        def copy_tile(rb, c, off):
            for dm in range(8):
                for k in range(8):
                    row_v[rb, c, dm, pl.ds(k * 16, 16)] = (
                        slab_v[pb, dm, pl.ds(off + c * 128 + k * 16, 16)])

        def assemble(m, rb):
            off = 8 * ((_M - 1) - m)

            def tiles(c, c2):
                copy_tile(rb, c, off)
                return c2

            lax.fori_loop(0, _NJT, tiles, 0)

        def assemble_inc(m, rb):
            off = 8 * ((_M - 1) - m)
            c_lo = jnp.clip(lax.div(w0_s - off - 127, 128), 0, _NJT - 5)

            def tiles(cc, c2):
                copy_tile(rb, c_lo + cc, off)
                return c2

            lax.fori_loop(0, 5, tiles, 0)

        def fire(m, rb):
            ri = (7 - s) + 8 * m
            pltpu.make_async_copy(
                row_v.at[rb], out_hbm.at[i0 + ri, dt], sem).start()

        def wait_one():
            pltpu.make_async_copy(
                row_v.at[0], out_hbm.at[i0, dt], sem).wait()

        assemble(0, 0)
        fire(0, 0)
        assemble(1, 1)
        fire(1, 1)

        def pump(m, c2):
            wait_one()
            rb = lax.rem(m, 2)
            assemble_inc(m, rb)
            fire(m, rb)
            return c2

        lax.fori_loop(2, _M, pump, 0)
        wait_one()
        wait_one()

        drain_stage()
        return carry

    stage(0, 0)
    drain_stage()
    lax.fori_loop(0, 8, phase, 0)


_rp_call = functools.partial(
    pl.kernel,
    mesh=plsc.VectorSubcoreMesh(core_axis_name="c", subcore_axis_name="s"),
    out_type=jax.ShapeDtypeStruct((_LQ, _NDT, _NJT, 8, 128), jnp.float32),
    scratch_types=[
        pltpu.VMEM((8 * _WCH, 128), jnp.int32),
        pltpu.VMEM((2, 8, _SPAN), jnp.float32),
        pltpu.VMEM((16,), jnp.int32),
        pltpu.VMEM((2, 128), jnp.float32),
        pltpu.VMEM((2, 128), jnp.int32),
        pltpu.VMEM((2, _NJT, 8, 128), jnp.float32),
        pltpu.SemaphoreType.DMA,
        pltpu.SemaphoreType.DMA,
    ],
    compiler_params=pltpu.CompilerParams(use_tc_tiling_on_sc=False),
)(_rp_body)


def kernel(length_q, length_k, embeddings_table):
    tbl = embeddings_table.astype(jnp.float32).reshape(_TFLAT)
    delta = jnp.zeros((16,), jnp.int32) + (
        jnp.asarray(length_k, jnp.int32) - jnp.asarray(length_q, jnp.int32))
    out5 = _rp_call(tbl, delta)
    return out5.transpose(0, 2, 4, 1, 3).reshape(_LQ, _LK, _D)


# speedup vs baseline: 2.1191x; 1.0085x over previous
import functools

import jax
import jax.numpy as jnp
from jax import lax
from jax.experimental import pallas as pl
from jax.experimental.pallas import tpu as pltpu
from jax.experimental.pallas import tpu_sc as plsc

_MAXP = 128
_D = 64
_LQ = 2048
_LK = 2048
_TFLAT = 257 * _D

_NDT = 8
_NJT = _LK // 128
_NIQ = 4
_IB = _LQ // _NIQ
_SPAN = _LK + _IB
_W = 384
_WCH = _W // 128
_M = _IB // 8


def _rp_body(table_hbm, delta_hbm, out_hbm, idx_v, slab_v, delta_v, fb_v,
             fbi_v, row_v, gsem, sem):
    wid = lax.axis_index("s") * 2 + lax.axis_index("c")
    dt = wid % _NDT
    iq = wid // _NDT
    d0 = dt * 8
    i0 = iq * _IB
    t_lo = (_LQ - _IB) - i0

    pltpu.sync_copy(delta_hbm, delta_v)
    delta = delta_v[...]
    base = t_lo - (_LQ - 1)
    lanes = lax.iota(jnp.int32, 16)

    zl = lanes * 0
    for dm in range(8):
        fbi_v[0, pl.ds(dm * 16, 16)] = zl + (d0 + dm)
        fbi_v[1, pl.ds(dm * 16, 16)] = zl + (256 * _D + d0 + dm)
    pltpu.make_async_copy(table_hbm.at[fbi_v.at[0]], fb_v.at[0], gsem).start()
    pltpu.make_async_copy(table_hbm.at[fbi_v.at[1]], fb_v.at[1], gsem).start()
    for _ in range(2):
        pltpu.make_async_copy(
            table_hbm.at[fbi_v.at[0]], fb_v.at[0], gsem).wait()
    fl = [fb_v[0, pl.ds(dm * 16, 16)] for dm in range(8)]
    fr = [fb_v[1, pl.ds(dm * 16, 16)] for dm in range(8)]

    def w0f(s):
        return 8 * jnp.clip(lax.div(-(base + s) - _MAXP - 60, 8),
                            0, (_SPAN - _W) // 8)

    def stage(s, pb):
        w0 = w0f(s)
        first = s <= 1
        lo = jnp.where(first, 0,
                       jnp.clip(lax.div(w0f(s - 2), 16), 0, _SPAN // 16 - 27))
        hi = jnp.where(first, _SPAN // 16, lo + 27)

        for dm in range(8):
            def fill(kk, c2, dm=dm):
                t = lanes + (kk * 16 + base + s) + delta
                val = jnp.where(t <= -_MAXP, fl[dm], fr[dm])
                slab_v[pb, dm, pl.ds(kk * 16, 16)] = val
                return c2

            lax.fori_loop(lo, hi, fill, 0)

        for dm in range(8):
            def fill_idx(q, c2, dm=dm):
                t = lanes + (q * 16 + w0 + base + s) + delta
                t = jnp.minimum(jnp.maximum(t, -_MAXP), _MAXP) + _MAXP
                idx_v[dm * _WCH + q // 8,
                      pl.ds((q % 8) * 16, 16)] = t * _D + d0 + dm
                return c2

            lax.fori_loop(0, _W // 16, fill_idx, 0)

        for g in range(8 * _WCH):
            pltpu.make_async_copy(
                table_hbm.at[idx_v.at[g]],
                slab_v.at[pb, g // _WCH, pl.ds(w0 + (g % _WCH) * 128, 128)],
                gsem,
            ).start()

    def drain_stage():
        for g in range(8 * _WCH):
            pltpu.make_async_copy(
                table_hbm.at[idx_v.at[0]],
                slab_v.at[0, 0, pl.ds(0, 128)],
                gsem,
            ).wait()

    def phase(s, carry):
        pb = lax.rem(s, 2)
        w0_s = w0f(s)
        stage(s + 1, 1 - pb)

        def copy_tile(rb, c, off):
            for dm in range(8):
                for k in range(8):
                    row_v[rb, c, dm, pl.ds(k * 16, 16)] = (
                        slab_v[pb, dm, pl.ds(off + c * 128 + k * 16, 16)])

        def assemble(m, rb):
            off = 8 * ((_M - 1) - m)

            def tiles(c, c2):
                copy_tile(rb, c, off)
                return c2

            lax.fori_loop(0, _NJT, tiles, 0)

        def assemble_inc(m, rb):
            off = 8 * ((_M - 1) - m)
            c_lo = jnp.clip(lax.div(w0_s - off - 127, 128), 0, _NJT - 5)

            def tiles(cc, c2):
                copy_tile(rb, c_lo + cc, off)
                return c2

            lax.fori_loop(0, 5, tiles, 0)

        def fire(m, rb):
            ri = (7 - s) + 8 * m
            pltpu.make_async_copy(
                row_v.at[rb], out_hbm.at[i0 + ri, dt], sem).start()

        def wait_one():
            pltpu.make_async_copy(
                row_v.at[0], out_hbm.at[i0, dt], sem).wait()

        for k in range(4):
            assemble(k, k)
            fire(k, k)

        def pump(m, c2):
            wait_one()
            rb = lax.rem(m, 4)
            assemble_inc(m, rb)
            fire(m, rb)
            return c2

        lax.fori_loop(4, _M, pump, 0)
        for k in range(4):
            wait_one()

        drain_stage()
        return carry

    stage(0, 0)
    drain_stage()
    lax.fori_loop(0, 8, phase, 0)


_rp_call = functools.partial(
    pl.kernel,
    mesh=plsc.VectorSubcoreMesh(core_axis_name="c", subcore_axis_name="s"),
    out_type=jax.ShapeDtypeStruct((_LQ, _NDT, _NJT, 8, 128), jnp.float32),
    scratch_types=[
        pltpu.VMEM((8 * _WCH, 128), jnp.int32),
        pltpu.VMEM((2, 8, _SPAN), jnp.float32),
        pltpu.VMEM((16,), jnp.int32),
        pltpu.VMEM((2, 128), jnp.float32),
        pltpu.VMEM((2, 128), jnp.int32),
        pltpu.VMEM((4, _NJT, 8, 128), jnp.float32),
        pltpu.SemaphoreType.DMA,
        pltpu.SemaphoreType.DMA,
    ],
    compiler_params=pltpu.CompilerParams(use_tc_tiling_on_sc=False),
)(_rp_body)


def kernel(length_q, length_k, embeddings_table):
    tbl = embeddings_table.astype(jnp.float32).reshape(_TFLAT)
    delta = jnp.zeros((16,), jnp.int32) + (
        jnp.asarray(length_k, jnp.int32) - jnp.asarray(length_q, jnp.int32))
    out5 = _rp_call(tbl, delta)
    return out5.transpose(0, 2, 4, 1, 3).reshape(_LQ, _LK, _D)
